# Initial kernel scaffold; baseline (speedup 1.0000x reference)
import jax
import jax.numpy as jnp
from jax.experimental import pallas as pl


def kernel(x, edge_index, edge_type, W1, root1, b1, W2, root2, b2):
    raise NotImplementedError("write your pallas kernel here")


# speedup vs baseline: 4.3323x; 4.3323x over previous
import functools

import jax
import jax.numpy as jnp
from jax import lax
from jax.experimental import pallas as pl
from jax.experimental.pallas import tpu as pltpu
from jax.experimental.pallas import tpu_sc as plsc

_N = 10000
_E = 320000
_D = 128
_DH = _D // 2
_R = 16
_NC = 2
_NS = 16
_CH = 80
_EPT = _E // _NS
_NCHUNK = _EPT // _CH
_EPW = _E // (_NC * _NS)
_NCHUNKW = _EPW // _CH
_RCH = 400
_NRC = _N // _RCH
_LG = _CH // 16

_mesh = plsc.VectorSubcoreMesh(core_axis_name="c", subcore_axis_name="s")


def _f32(*shape):
    return jax.ShapeDtypeStruct(shape, jnp.float32)


@functools.partial(
    pl.kernel,
    out_type=_f32(_NC * _N * _R),
    mesh=_mesh,
    scratch_types=[
        pltpu.VMEM((_CH,), jnp.int32),
        pltpu.VMEM((_CH,), jnp.float32),
        pltpu.VMEM(((_N * _R) // _NS,), jnp.float32),
        pltpu.VMEM_SHARED((_N * _R,), jnp.float32),
    ],
)
def _count_k(fidx_hbm, ones_hbm, cnt_hbm, idx_v, ones_v, stg_v, cnt_sh):
    cid = lax.axis_index("c")
    sid = lax.axis_index("s")
    wid = cid * _NS + sid
    seg = (_N * _R) // _NS

    def zbody(i, carry):
        stg_v[pl.ds(i * 16, 16)] = jnp.zeros((16,), jnp.float32)
        return carry

    lax.fori_loop(0, seg // 16, zbody, 0)
    pltpu.sync_copy(stg_v, cnt_sh.at[pl.ds(sid * seg, seg)])
    pltpu.sync_copy(ones_hbm, ones_v)
    plsc.subcore_barrier()
    base = wid * _EPW

    def body(i, carry):
        pltpu.sync_copy(fidx_hbm.at[pl.ds(base + i * _CH, _CH)], idx_v)
        pltpu.sync_copy(ones_v, cnt_sh.at[idx_v], add=True)
        return carry

    lax.fori_loop(0, _NCHUNKW, body, 0)
    plsc.subcore_barrier()
    pltpu.sync_copy(cnt_sh.at[pl.ds(sid * seg, seg)], stg_v)
    pltpu.sync_copy(stg_v, cnt_hbm.at[pl.ds(cid * (_N * _R) + sid * seg, seg)])


@functools.partial(
    pl.kernel,
    out_type=_f32(_E),
    mesh=_mesh,
    scratch_types=[
        pltpu.VMEM((_CH,), jnp.int32),
        pltpu.VMEM((_CH,), jnp.int32),
        pltpu.VMEM((_CH,), jnp.float32),
        pltpu.VMEM((_CH,), jnp.float32),
        pltpu.VMEM((_CH,), jnp.float32),
        pltpu.SemaphoreType.DMA,
    ],
)
def _norm_k(fidx_hbm, cntf_hbm, nrm_hbm, idx_v, idx2_v, c0_v, c1_v, nv, sem):
    cid = lax.axis_index("c")
    sid = lax.axis_index("s")
    wid = cid * _NS + sid
    base = wid * _EPW

    def body(i, carry):
        pltpu.sync_copy(fidx_hbm.at[pl.ds(base + i * _CH, _CH)], idx_v)
        for j in range(_LG):
            idx2_v[pl.ds(j * 16, 16)] = idx_v[pl.ds(j * 16, 16)] + (_N * _R)
        pltpu.async_copy(cntf_hbm.at[idx_v], c0_v, sem).wait()
        pltpu.async_copy(cntf_hbm.at[idx2_v], c1_v, sem).wait()
        for j in range(_LG):
            c = c0_v[pl.ds(j * 16, 16)] + c1_v[pl.ds(j * 16, 16)]
            nv[pl.ds(j * 16, 16)] = 1.0 / jnp.maximum(c, 1.0)
        pltpu.sync_copy(nv, nrm_hbm.at[pl.ds(base + i * _CH, _CH)])
        return carry

    lax.fori_loop(0, _NCHUNKW, body, 0)


@functools.partial(
    pl.kernel,
    out_type=_f32(_NC, _N, _DH),
    mesh=_mesh,
    scratch_types=[
        pltpu.VMEM((_CH,), jnp.int32),
        pltpu.VMEM((_CH,), jnp.int32),
        pltpu.VMEM((_CH,), jnp.float32),
        pltpu.VMEM((_CH, _DH), jnp.float32),
        pltpu.VMEM((_RCH, _DH), jnp.float32),
        pltpu.VMEM_SHARED((_N, _DH), jnp.float32),
        pltpu.SemaphoreType.DMA,
    ],
    compiler_params=pltpu.CompilerParams(use_tc_tiling_on_sc=False),
)
def _scatter_k(xw_hbm, gidx_hbm, dst_hbm, nrm_hbm, zrow_hbm, pagg_hbm,
               gidx_v, dst_v, nrm_v, rows_v, stg_v, acc_sh, sem):
    cid = lax.axis_index("c")
    sid = lax.axis_index("s")
    pltpu.sync_copy(zrow_hbm, stg_v)
    pltpu.sync_copy(stg_v, acc_sh.at[pl.ds(sid * _RCH, _RCH)])

    @pl.when(sid < _NRC - _NS)
    def _():
        pltpu.sync_copy(stg_v, acc_sh.at[pl.ds((sid + _NS) * _RCH, _RCH)])

    plsc.subcore_barrier()
    base = sid * _EPT
    xw_half = xw_hbm.at[cid]

    def body(i, carry):
        pltpu.sync_copy(gidx_hbm.at[pl.ds(base + i * _CH, _CH)], gidx_v)
        pltpu.sync_copy(dst_hbm.at[pl.ds(base + i * _CH, _CH)], dst_v)
        pltpu.sync_copy(nrm_hbm.at[pl.ds(base + i * _CH, _CH)], nrm_v)
        pltpu.async_copy(xw_half.at[gidx_v], rows_v, sem).wait()

        for g in range(_LG):
            nv16 = nrm_v[pl.ds(g * 16, 16)]
            for l in range(16):
                e = g * 16 + l
                sv = jnp.full((16,), nv16[l], jnp.float32)
                for j in range(_DH // 16):
                    rows_v[e, pl.ds(j * 16, 16)] = (
                        rows_v[e, pl.ds(j * 16, 16)] * sv)

        pltpu.sync_copy(rows_v, acc_sh.at[dst_v], add=True)
        return carry

    lax.fori_loop(0, _NCHUNK, body, 0)
    plsc.subcore_barrier()

    def out_chunk(c):
        pltpu.sync_copy(acc_sh.at[pl.ds(c * _RCH, _RCH)], stg_v)
        pltpu.sync_copy(stg_v, pagg_hbm.at[cid, pl.ds(c * _RCH, _RCH)])

    out_chunk(sid)

    @pl.when(sid < _NRC - _NS)
    def _():
        out_chunk(sid + _NS)


_NB = 10
_BN = _N // _NB


def _rel_matmul(h, W):
    def mm(h_ref, w_ref, o_ref):
        res = jnp.dot(h_ref[...], w_ref[0], preferred_element_type=jnp.float32)
        o_ref[0, 0] = res[:, :_DH]
        o_ref[1, 0] = res[:, _DH:]

    return pl.pallas_call(
        mm,
        grid=(_NB, _R),
        in_specs=[
            pl.BlockSpec((_BN, _D), lambda i, r: (i, 0)),
            pl.BlockSpec((1, _D, _D), lambda i, r: (r, 0, 0)),
        ],
        out_specs=pl.BlockSpec((_NC, 1, _BN, _DH), lambda i, r: (0, r, i, 0)),
        out_shape=_f32(_NC, _R, _N, _DH),
    )(h, W)


def _combine(pagg, h, root, b2d, relu):
    def comb(p_ref, h_ref, root_ref, b_ref, o_ref):
        v = jnp.concatenate([p_ref[0], p_ref[1]], axis=-1)
        v = v + jnp.dot(h_ref[...], root_ref[...],
                        preferred_element_type=jnp.float32) + b_ref[...]
        if relu:
            v = jnp.maximum(v, 0.0)
        o_ref[...] = v

    return pl.pallas_call(
        comb,
        grid=(_NB,),
        in_specs=[
            pl.BlockSpec((_NC, _BN, _DH), lambda i: (0, i, 0)),
            pl.BlockSpec((_BN, _D), lambda i: (i, 0)),
            pl.BlockSpec((_D, _D), lambda i: (0, 0)),
            pl.BlockSpec((1, _D), lambda i: (0, 0)),
        ],
        out_specs=pl.BlockSpec((_BN, _D), lambda i: (i, 0)),
        out_shape=_f32(_N, _D),
    )(pagg, h, root, b2d)


def _layer(h, W, root, b2d, gidx, dst, nrm, zrow, relu):
    xw = _rel_matmul(h, W).reshape(_NC, _R * _N, _DH)
    pagg = _scatter_k(xw, gidx, dst, nrm, zrow)
    return _combine(pagg, h, root, b2d, relu)


def kernel(x, edge_index, edge_type, W1, root1, b1, W2, root2, b2):
    src = edge_index[0].astype(jnp.int32)
    dst = edge_index[1].astype(jnp.int32)
    rel = edge_type.astype(jnp.int32)
    fidx = dst * _R + rel
    gidx = rel * _N + src
    ones = jnp.ones((_CH,), jnp.float32)
    zrow = jnp.zeros((_RCH, _DH), jnp.float32)

    cnt = _count_k(fidx, ones)
    nrm = _norm_k(fidx, cnt)

    h = _layer(x, W1, root1, b1.reshape(1, _D), gidx, dst, nrm, zrow, True)
    out = _layer(h, W2, root2, b2.reshape(1, _D), gidx, dst, nrm, zrow, False)
    return out


# speedup vs baseline: 5.6877x; 1.3129x over previous
import functools

import jax
import jax.numpy as jnp
from jax import lax
from jax.experimental import pallas as pl
from jax.experimental.pallas import tpu as pltpu
from jax.experimental.pallas import tpu_sc as plsc

_N = 10000
_E = 320000
_D = 128
_DH = _D // 2
_R = 16
_NC = 2
_NS = 16
_CH = 80
_NCHT = (_E // _NS) // _CH
_NCHW = (_E // (_NC * _NS)) // _CH
_RCH = 400
_NRC = _N // _RCH
_LG = _CH // 16

_mesh = plsc.VectorSubcoreMesh(core_axis_name="c", subcore_axis_name="s")
_sc_params = pltpu.CompilerParams(use_tc_tiling_on_sc=False)
_sc_params_nl = pltpu.CompilerParams(use_tc_tiling_on_sc=False,
                                    needs_layout_passes=False)


def _f32(*shape):
    return jax.ShapeDtypeStruct(shape, jnp.float32)


_CROW = (_N * _R) // 16
_CRPT = _CROW // _NS


@functools.partial(
    pl.kernel,
    out_type=_f32(_NC * _CROW, 16),
    mesh=_mesh,
    scratch_types=[
        pltpu.VMEM((_NCHW, _CH), jnp.int32),
        pltpu.VMEM((_CH,), jnp.int32),
        pltpu.VMEM((_CH, 16), jnp.float32),
        pltpu.VMEM((_CRPT, 16), jnp.float32),
        pltpu.VMEM_SHARED((_CROW, 16), jnp.float32),
    ],
    compiler_params=_sc_params,
)
def _count_k(fidx_hbm, zcnt_hbm, cnt_hbm, fidx_t, row1d, oh_v, stg_v, cnt_sh):
    cid = lax.axis_index("c")
    sid = lax.axis_index("s")
    wid = cid * _NS + sid
    pltpu.sync_copy(fidx_hbm.at[wid], fidx_t)
    pltpu.sync_copy(zcnt_hbm, stg_v)
    pltpu.sync_copy(stg_v, cnt_sh.at[pl.ds(sid * _CRPT, _CRPT)])
    plsc.subcore_barrier()

    lane = lax.iota(jnp.int32, 16)
    onef = jnp.ones((16,), jnp.float32)
    zerof = jnp.zeros((16,), jnp.float32)

    def body(i, carry):
        for g in range(_LG):
            fv = fidx_t[i, pl.ds(g * 16, 16)]
            row1d[pl.ds(g * 16, 16)] = fv >> 4
            hotv = fv & 15
            for l in range(16):
                hv = jnp.full((16,), hotv[l], jnp.int32)
                oh_v[g * 16 + l, pl.ds(0, 16)] = jnp.where(
                    lane == hv, onef, zerof)
        pltpu.sync_copy(oh_v, cnt_sh.at[row1d], add=True)
        return carry

    lax.fori_loop(0, _NCHW, body, 0)
    plsc.subcore_barrier()
    pltpu.sync_copy(cnt_sh.at[pl.ds(sid * _CRPT, _CRPT)], stg_v)
    pltpu.sync_copy(stg_v, cnt_hbm.at[pl.ds(cid * _CROW + sid * _CRPT,
                                            _CRPT)])


@functools.partial(
    pl.kernel,
    out_type=_f32(_NC * _NS, _NCHW, _CH),
    mesh=_mesh,
    scratch_types=[
        pltpu.VMEM((_NCHW, _CH), jnp.int32),
        pltpu.VMEM((_NCHW, _CH), jnp.int32),
        pltpu.VMEM((_CH,), jnp.float32),
        pltpu.VMEM((_CH,), jnp.float32),
        pltpu.VMEM((_CH,), jnp.float32),
        pltpu.VMEM((_CH,), jnp.float32),
        pltpu.VMEM((_NCHW, _CH), jnp.float32),
        pltpu.SemaphoreType.DMA,
        pltpu.SemaphoreType.DMA,
    ],
    compiler_params=_sc_params,
)
def _norm_k(fidx_hbm, cntf_hbm, nrm_hbm,
            fidx_t, fidx2_t, c0a_v, c1a_v, c0b_v, c1b_v, nrm_t, sem0, sem1):
    cid = lax.axis_index("c")
    sid = lax.axis_index("s")
    wid = cid * _NS + sid
    pltpu.sync_copy(fidx_hbm.at[wid], fidx_t)

    def off(i, carry):
        def offg(j, carry2):
            fidx2_t[i, pl.ds(j * 16, 16)] = (
                fidx_t[i, pl.ds(j * 16, 16)] + (_N * _R))
            return carry2
        return lax.fori_loop(0, _LG, offg, carry)

    lax.fori_loop(0, _NCHW, off, 0)

    def compute(i, c0_v, c1_v):
        def cg(j, carry):
            c = c0_v[pl.ds(j * 16, 16)] + c1_v[pl.ds(j * 16, 16)]
            nrm_t[i, pl.ds(j * 16, 16)] = 1.0 / jnp.maximum(c, 1.0)
            return carry
        lax.fori_loop(0, _LG, cg, 0)

    pltpu.async_copy(cntf_hbm.at[fidx_t.at[0]], c0a_v, sem0)
    pltpu.make_async_copy(cntf_hbm.at[fidx_t.at[0]], c0a_v, sem0).wait()
    pltpu.async_copy(cntf_hbm.at[fidx2_t.at[0]], c1a_v, sem0)
    pltpu.make_async_copy(cntf_hbm.at[fidx2_t.at[0]], c1a_v, sem0).wait()
    compute(0, c0a_v, c1a_v)

    pltpu.async_copy(cntf_hbm.at[fidx_t.at[1]], c0a_v, sem0)
    pltpu.async_copy(cntf_hbm.at[fidx2_t.at[1]], c1a_v, sem0)
    pltpu.async_copy(cntf_hbm.at[fidx_t.at[2]], c0b_v, sem1)
    pltpu.async_copy(cntf_hbm.at[fidx2_t.at[2]], c1b_v, sem1)

    def body(t, carry):
        a = 2 * t + 1
        b = a + 1
        pltpu.make_async_copy(cntf_hbm.at[fidx_t.at[a]], c0a_v, sem0).wait()
        pltpu.make_async_copy(cntf_hbm.at[fidx2_t.at[a]], c1a_v, sem0).wait()
        compute(a, c0a_v, c1a_v)
        pltpu.async_copy(cntf_hbm.at[fidx_t.at[a + 2]], c0a_v, sem0)
        pltpu.async_copy(cntf_hbm.at[fidx2_t.at[a + 2]], c1a_v, sem0)
        pltpu.make_async_copy(cntf_hbm.at[fidx_t.at[b]], c0b_v, sem1).wait()
        pltpu.make_async_copy(cntf_hbm.at[fidx2_t.at[b]], c1b_v, sem1).wait()
        compute(b, c0b_v, c1b_v)
        pltpu.async_copy(cntf_hbm.at[fidx_t.at[b + 2]], c0b_v, sem1)
        pltpu.async_copy(cntf_hbm.at[fidx2_t.at[b + 2]], c1b_v, sem1)
        return carry

    lax.fori_loop(0, (_NCHW - 1) // 2 - 1, body, 0)
    a = _NCHW - 2
    b = _NCHW - 1
    pltpu.make_async_copy(cntf_hbm.at[fidx_t.at[a]], c0a_v, sem0).wait()
    pltpu.make_async_copy(cntf_hbm.at[fidx2_t.at[a]], c1a_v, sem0).wait()
    compute(a, c0a_v, c1a_v)
    pltpu.make_async_copy(cntf_hbm.at[fidx_t.at[b]], c0b_v, sem1).wait()
    pltpu.make_async_copy(cntf_hbm.at[fidx2_t.at[b]], c1b_v, sem1).wait()
    compute(b, c0b_v, c1b_v)
    pltpu.sync_copy(nrm_t, nrm_hbm.at[wid])


_DQ = _D // 4


@functools.partial(
    pl.kernel,
    out_type=_f32(4, _N, _DQ),
    mesh=_mesh,
    scratch_types=[
        pltpu.VMEM((_NCHT, _CH), jnp.int32),
        pltpu.VMEM((_NCHT, _CH), jnp.int32),
        pltpu.VMEM((_NCHT, _CH), jnp.float32),
        pltpu.VMEM((_CH, _DQ), jnp.float32),
        pltpu.VMEM((_CH, _DQ), jnp.float32),
        pltpu.VMEM((_RCH, _DQ), jnp.float32),
        pltpu.VMEM_SHARED((_N, _DQ), jnp.float32),
        pltpu.SemaphoreType.DMA,
        pltpu.SemaphoreType.DMA,
    ],
    compiler_params=_sc_params,
)
def _scatter_k(xw_hbm, gidx_hbm, dst_hbm, nrm_hbm, zrow_hbm, pagg_hbm,
               gidx_t, dst_t, nrm_t, rows0_v, rows1_v, stg_v, acc_sh,
               sem0, sem1):
    cid = lax.axis_index("c")
    sid = lax.axis_index("s")
    pltpu.sync_copy(gidx_hbm.at[sid], gidx_t)
    pltpu.sync_copy(dst_hbm.at[sid], dst_t)
    pltpu.sync_copy(nrm_hbm.at[sid], nrm_t)

    def process(k, rows_v, xw_q):
        for g in range(_LG):
            nv16 = nrm_t[k, pl.ds(g * 16, 16)]
            for l in range(16):
                e = g * 16 + l
                sv = jnp.full((16,), nv16[l], jnp.float32)
                for j in range(_DQ // 16):
                    rows_v[e, pl.ds(j * 16, 16)] = (
                        rows_v[e, pl.ds(j * 16, 16)] * sv)
        pltpu.sync_copy(rows_v, acc_sh.at[dst_t.at[k]], add=True)

    def one_pass(p, carry):
        q = cid * 2 + p
        xw_q = xw_hbm.at[q]
        pltpu.sync_copy(zrow_hbm, stg_v)
        pltpu.sync_copy(stg_v, acc_sh.at[pl.ds(sid * _RCH, _RCH)])

        @pl.when(sid < _NRC - _NS)
        def _():
            pltpu.sync_copy(stg_v, acc_sh.at[pl.ds((sid + _NS) * _RCH, _RCH)])

        plsc.subcore_barrier()

        pltpu.async_copy(xw_q.at[gidx_t.at[0]], rows0_v, sem0)
        pltpu.async_copy(xw_q.at[gidx_t.at[1]], rows1_v, sem1)

        def body(t, carry2):
            a = 2 * t
            b = a + 1
            pltpu.make_async_copy(xw_q.at[gidx_t.at[a]], rows0_v, sem0).wait()
            process(a, rows0_v, xw_q)
            pltpu.async_copy(xw_q.at[gidx_t.at[a + 2]], rows0_v, sem0)
            pltpu.make_async_copy(xw_q.at[gidx_t.at[b]], rows1_v, sem1).wait()
            process(b, rows1_v, xw_q)
            pltpu.async_copy(xw_q.at[gidx_t.at[b + 2]], rows1_v, sem1)
            return carry2

        lax.fori_loop(0, _NCHT // 2 - 1, body, 0)
        a = _NCHT - 2
        b = _NCHT - 1
        pltpu.make_async_copy(xw_q.at[gidx_t.at[a]], rows0_v, sem0).wait()
        process(a, rows0_v, xw_q)
        pltpu.make_async_copy(xw_q.at[gidx_t.at[b]], rows1_v, sem1).wait()
        process(b, rows1_v, xw_q)
        plsc.subcore_barrier()

        def out_chunk(c):
            pltpu.sync_copy(acc_sh.at[pl.ds(c * _RCH, _RCH)], stg_v)
            pltpu.sync_copy(stg_v, pagg_hbm.at[q, pl.ds(c * _RCH, _RCH)])

        out_chunk(sid)

        @pl.when(sid < _NRC - _NS)
        def _():
            out_chunk(sid + _NS)

        plsc.subcore_barrier()
        return carry

    lax.fori_loop(0, 2, one_pass, 0)


_NB = 10
_BN = _N // _NB


def _rel_matmul(h, W):
    def mm(h_ref, w_ref, o_ref):
        res = jnp.dot(h_ref[...], w_ref[0], preferred_element_type=jnp.float32)
        for q in range(4):
            o_ref[q, 0] = res[:, q * _DQ:(q + 1) * _DQ]

    return pl.pallas_call(
        mm,
        grid=(_NB, _R),
        in_specs=[
            pl.BlockSpec((_BN, _D), lambda i, r: (i, 0)),
            pl.BlockSpec((1, _D, _D), lambda i, r: (r, 0, 0)),
        ],
        out_specs=pl.BlockSpec((4, 1, _BN, _DQ), lambda i, r: (0, r, i, 0)),
        out_shape=_f32(4, _R, _N, _DQ),
    )(h, W)


def _combine(pagg, h, root, b2d, flag):
    def comb(flag_ref, p_ref, h_ref, root_ref, b_ref, o_ref):
        v = jnp.concatenate([p_ref[0], p_ref[1], p_ref[2], p_ref[3]],
                            axis=-1)
        v = v + jnp.dot(h_ref[...], root_ref[...],
                        preferred_element_type=jnp.float32) + b_ref[...]
        o_ref[...] = jnp.where(flag_ref[0] > 0.5, jnp.maximum(v, 0.0), v)

    return pl.pallas_call(
        comb,
        grid=(_NB,),
        in_specs=[
            pl.BlockSpec(memory_space=pltpu.SMEM),
            pl.BlockSpec((4, _BN, _DQ), lambda i: (0, i, 0)),
            pl.BlockSpec((_BN, _D), lambda i: (i, 0)),
            pl.BlockSpec((_D, _D), lambda i: (0, 0)),
            pl.BlockSpec((1, _D), lambda i: (0, 0)),
        ],
        out_specs=pl.BlockSpec((_BN, _D), lambda i: (i, 0)),
        out_shape=_f32(_N, _D),
    )(flag, pagg, h, root, b2d)


def kernel(x, edge_index, edge_type, W1, root1, b1, W2, root2, b2):
    src = edge_index[0].astype(jnp.int32)
    dst = edge_index[1].astype(jnp.int32)
    rel = edge_type.astype(jnp.int32)
    fidx = dst * _R + rel
    fidx3n = fidx.reshape(_NC * _NS, _NCHW, _CH)
    gidx3 = (rel * _N + src).reshape(_NS, _NCHT, _CH)
    dst3 = dst.reshape(_NS, _NCHT, _CH)
    zrow = jnp.zeros((_RCH, _DQ), jnp.float32)

    zcnt = jnp.zeros((_CRPT, 16), jnp.float32)
    cnt = _count_k(fidx3n, zcnt).reshape(_NC * _N * _R)
    nrm3 = _norm_k(fidx3n, cnt).reshape(_NS, _NCHT, _CH)

    Ws = jnp.stack([W1, W2])
    roots = jnp.stack([root1, root2])
    bs = jnp.stack([b1.reshape(1, _D), b2.reshape(1, _D)])
    flags = jnp.array([[1.0], [0.0]], jnp.float32)

    def step(h, xs):
        W, root, b2d, flag = xs
        xw = _rel_matmul(h, W).reshape(4, _R * _N, _DQ)
        pagg = _scatter_k(xw, gidx3, dst3, nrm3, zrow)
        return _combine(pagg, h, root, b2d, flag), None

    out, _ = lax.scan(step, x, (Ws, roots, bs, flags))
    return out
